# baseline (speedup 1.0000x reference)
import functools

import jax
import jax.numpy as jnp
from jax import lax
from jax.experimental import pallas as pl
from jax.experimental.pallas import tpu as pltpu
from jax.experimental.pallas import tpu_sc as plsc

_CH = 64
_CHS = 6


def kernel(out0, out1, out2, out3, comp_id):
    B, D = out0.shape
    info = plsc.get_sparse_core_info()
    NC, NS, L = info.num_cores, info.num_subcores, info.num_lanes
    NW = NC * NS
    bpw = B // NW
    ngrp = bpw // L
    CH = _CH
    SEGF = bpw + CH
    cid1d = comp_id.reshape(B)

    mesh = plsc.VectorSubcoreMesh(core_axis_name="c", subcore_axis_name="s")

    @functools.partial(
        pl.kernel,
        mesh=mesh,
        out_type=jax.ShapeDtypeStruct((B, D), jnp.float32),
        scratch_types=[
            pltpu.VMEM((bpw,), jnp.int32),
            pltpu.VMEM((4 * SEGF,), jnp.int32),
            pltpu.VMEM((L,), jnp.int32),
            pltpu.VMEM((bpw + 4 * CH, D), jnp.float32),
            pltpu.SemaphoreType.DMA,
            pltpu.SemaphoreType.DMA,
            pltpu.SemaphoreType.DMA,
        ],
        compiler_params=pltpu.CompilerParams(needs_layout_passes=False),
    )
    def run(o0, o1, o2, o3, cid_hbm, out_hbm,
            cid_v, ilist, cnt_v, rows_v, gsemA, gsemB, ssem):
        tables = (o0, o1, o2, o3)
        wid = lax.axis_index("s") * NC + lax.axis_index("c")
        base = wid * bpw
        pltpu.sync_copy(cid_hbm.at[pl.ds(base, bpw)], cid_v)

        cnt_v[pl.ds(0, L)] = jnp.zeros((L,), jnp.int32)
        iota = lax.iota(jnp.int32, L)
        basis = plsc.scan_count(jnp.zeros((L,), jnp.int32))[0][0]

        def grp_body(g, carry):
            cid16 = cid_v[pl.ds(g * L, L)]
            rows16 = (base + g * L) + iota
            dup, lastm = plsc.scan_count(cid16)
            cntv = plsc.load_gather(cnt_v, [cid16])
            rel = cntv + (dup - basis)
            plsc.store_scatter(ilist, [cid16 * SEGF + rel], rows16)
            plsc.store_scatter(cnt_v, [cid16], rel + 1, mask=lastm)
            return carry

        lax.fori_loop(0, ngrp, grp_body, 0)

        cvec = cnt_v[pl.ds(0, L)]
        cnts = tuple(cvec[k] for k in range(4))
        nch = tuple((cnts[k] + (CH - 1)) // CH for k in range(4))
        offs = (jnp.int32(0), nch[0] * CH, (nch[0] + nch[1]) * CH,
                (nch[0] + nch[1] + nch[2]) * CH)

        for k in range(4):
            last_at = k * SEGF + jnp.maximum(cnts[k] - 1, 0)
            padv = plsc.load_gather(
                ilist, [jnp.full((L,), last_at, jnp.int32)])
            for i in range(CH // L):
                ilist[pl.ds(k * SEGF + cnts[k] + i * L, L)] = padv

        def g_copy(k, j, sem):
            return pltpu.make_async_copy(
                tables[k].at[ilist.at[pl.ds(k * SEGF + j * CH, CH)]],
                rows_v.at[pl.ds(offs[k] + j * CH, CH), :],
                sem)

        def s_copy(k, j):
            return pltpu.make_async_copy(
                rows_v.at[pl.ds(offs[k] + j * CH, CH), :],
                out_hbm.at[ilist.at[pl.ds(k * SEGF + j * CH, CH)]],
                ssem)

        gsems = (gsemA, gsemB)

        def g_fire(k):
            sem = gsems[k % 2]
            lax.fori_loop(
                0, nch[k],
                lambda j, _, k=k, sem=sem: (g_copy(k, j, sem).start(), 0)[1], 0)

        def g_drain(k):
            sem = gsems[k % 2]
            lax.fori_loop(
                0, nch[k],
                lambda j, _, k=k, sem=sem: (g_copy(k, j, sem).wait(), 0)[1], 0)

        def s_fire(k):
            lax.fori_loop(
                0, nch[k], lambda j, _, k=k: (s_copy(k, j).start(), 0)[1], 0)

        def s_drain(k):
            lax.fori_loop(
                0, nch[k], lambda j, _, k=k: (s_copy(k, j).wait(), 0)[1], 0)

        g_fire(0)
        g_fire(1)
        for k in range(4):
            g_drain(k)
            if k + 2 < 4:
                g_fire(k + 2)
            s_fire(k)
        for k in range(4):
            s_drain(k)

    return run(out0, out1, out2, out3, cid1d)


# speedup vs baseline: 1.5341x; 1.5341x over previous
import functools

import jax
import jax.numpy as jnp
from jax import lax
from jax.experimental import pallas as pl
from jax.experimental.pallas import tpu as pltpu
from jax.experimental.pallas import tpu_sc as plsc


def kernel(out0, out1, out2, out3, comp_id):
    B, D = out0.shape
    info = plsc.get_sparse_core_info()
    NC, NS, L = info.num_cores, info.num_subcores, info.num_lanes
    NW = NC * NS
    bpw = B // NW
    ngrp = bpw // L
    CH = L
    SEGF = bpw + CH
    cid1d = comp_id.reshape(B)

    mesh = plsc.VectorSubcoreMesh(core_axis_name="c", subcore_axis_name="s")

    @functools.partial(
        pl.kernel,
        mesh=mesh,
        out_type=jax.ShapeDtypeStruct((B, D), jnp.float32),
        scratch_types=[
            pltpu.VMEM((bpw,), jnp.int32),
            pltpu.VMEM((4 * SEGF,), jnp.int32),
            pltpu.VMEM((L,), jnp.int32),
            pltpu.VMEM((bpw + 4 * CH, D), jnp.float32),
            pltpu.SemaphoreType.DMA,
            pltpu.SemaphoreType.DMA,
            pltpu.SemaphoreType.DMA,
        ],
        compiler_params=pltpu.CompilerParams(needs_layout_passes=False),
    )
    def run(o0, o1, o2, o3, cid_hbm, out_hbm,
            cid_v, ilist, cnt_v, rows_v, gsemA, gsemB, ssem):
        tables = (o0, o1, o2, o3)
        wid = lax.axis_index("s") * NC + lax.axis_index("c")
        base = wid * bpw
        pltpu.sync_copy(cid_hbm.at[pl.ds(base, bpw)], cid_v)

        cnt_v[pl.ds(0, L)] = jnp.zeros((L,), jnp.int32)
        iota = lax.iota(jnp.int32, L)
        basis = plsc.scan_count(jnp.zeros((L,), jnp.int32))[0][0]

        def grp_body(g, carry):
            cid16 = cid_v[pl.ds(g * L, L)]
            rows16 = (base + g * L) + iota
            dup, lastm = plsc.scan_count(cid16)
            cntv = plsc.load_gather(cnt_v, [cid16])
            rel = cntv + (dup - basis)
            plsc.store_scatter(ilist, [cid16 * SEGF + rel], rows16)
            plsc.store_scatter(cnt_v, [cid16], rel + 1, mask=lastm)
            return carry

        lax.fori_loop(0, ngrp, grp_body, 0)

        cvec = cnt_v[pl.ds(0, L)]
        cnts = tuple(cvec[k] for k in range(4))
        nch = tuple((cnts[k] + (CH - 1)) // CH for k in range(4))
        offs = (jnp.int32(0), nch[0] * CH, (nch[0] + nch[1]) * CH,
                (nch[0] + nch[1] + nch[2]) * CH)

        for k in range(4):
            last_at = k * SEGF + jnp.maximum(cnts[k] - 1, 0)
            padv = plsc.load_gather(
                ilist, [jnp.full((L,), last_at, jnp.int32)])
            for i in range(CH // L):
                ilist[pl.ds(k * SEGF + cnts[k] + i * L, L)] = padv

        def g_copy(k, j):
            idx16 = ilist[pl.ds(k * SEGF + j * CH, CH)]
            return pltpu.make_async_copy(
                tables[k].at[idx16],
                rows_v.at[pl.ds(offs[k] + j * CH, CH), :],
                gsemA)

        def s_copy(k, j):
            idx16 = ilist[pl.ds(k * SEGF + j * CH, CH)]
            return pltpu.make_async_copy(
                rows_v.at[pl.ds(offs[k] + j * CH, CH), :],
                out_hbm.at[idx16],
                ssem)

        for k in range(4):
            lax.fori_loop(
                0, nch[k], lambda j, _, k=k: (g_copy(k, j).start(), 0)[1], 0)
        for k in range(4):
            lax.fori_loop(
                0, nch[k], lambda j, _, k=k: (g_copy(k, j).wait(), 0)[1], 0)
        for k in range(4):
            lax.fori_loop(
                0, nch[k], lambda j, _, k=k: (s_copy(k, j).start(), 0)[1], 0)
        for k in range(4):
            lax.fori_loop(
                0, nch[k], lambda j, _, k=k: (s_copy(k, j).wait(), 0)[1], 0)

    return run(out0, out1, out2, out3, cid1d)


# speedup vs baseline: 1.5378x; 1.0025x over previous
import functools

import jax
import jax.numpy as jnp
from jax import lax
from jax.experimental import pallas as pl
from jax.experimental.pallas import tpu as pltpu
from jax.experimental.pallas import tpu_sc as plsc


def kernel(out0, out1, out2, out3, comp_id):
    B, D = out0.shape
    info = plsc.get_sparse_core_info()
    NC, NS, L = info.num_cores, info.num_subcores, info.num_lanes
    NW = NC * NS
    bpw = B // NW
    ngrp = bpw // L
    CH = L
    SEGF = bpw + CH
    cid1d = comp_id.reshape(B)

    mesh = plsc.VectorSubcoreMesh(core_axis_name="c", subcore_axis_name="s")

    @functools.partial(
        pl.kernel,
        mesh=mesh,
        out_type=jax.ShapeDtypeStruct((B, D), jnp.float32),
        scratch_types=[
            pltpu.VMEM((bpw,), jnp.int32),
            pltpu.VMEM((4 * SEGF,), jnp.int32),
            pltpu.VMEM((L,), jnp.int32),
            pltpu.VMEM((bpw + 4 * CH, D), jnp.float32),
            pltpu.SemaphoreType.DMA,
            pltpu.SemaphoreType.DMA,
            pltpu.SemaphoreType.DMA,
        ],
        compiler_params=pltpu.CompilerParams(needs_layout_passes=False, use_tc_tiling_on_sc=False),
    )
    def run(o0, o1, o2, o3, cid_hbm, out_hbm,
            cid_v, ilist, cnt_v, rows_v, gsemA, gsemB, ssem):
        tables = (o0, o1, o2, o3)
        wid = lax.axis_index("s") * NC + lax.axis_index("c")
        base = wid * bpw
        pltpu.sync_copy(cid_hbm.at[pl.ds(base, bpw)], cid_v)

        cnt_v[pl.ds(0, L)] = jnp.zeros((L,), jnp.int32)
        iota = lax.iota(jnp.int32, L)
        basis = plsc.scan_count(jnp.zeros((L,), jnp.int32))[0][0]

        def grp_body(g, carry):
            cid16 = cid_v[pl.ds(g * L, L)]
            rows16 = (base + g * L) + iota
            dup, lastm = plsc.scan_count(cid16)
            cntv = plsc.load_gather(cnt_v, [cid16])
            rel = cntv + (dup - basis)
            plsc.store_scatter(ilist, [cid16 * SEGF + rel], rows16)
            plsc.store_scatter(cnt_v, [cid16], rel + 1, mask=lastm)
            return carry

        lax.fori_loop(0, ngrp, grp_body, 0)

        cvec = cnt_v[pl.ds(0, L)]
        cnts = tuple(cvec[k] for k in range(4))
        nch = tuple((cnts[k] + (CH - 1)) // CH for k in range(4))
        offs = (jnp.int32(0), nch[0] * CH, (nch[0] + nch[1]) * CH,
                (nch[0] + nch[1] + nch[2]) * CH)

        for k in range(4):
            last_at = k * SEGF + jnp.maximum(cnts[k] - 1, 0)
            padv = plsc.load_gather(
                ilist, [jnp.full((L,), last_at, jnp.int32)])
            for i in range(CH // L):
                ilist[pl.ds(k * SEGF + cnts[k] + i * L, L)] = padv

        def g_copy(k, j):
            idx16 = ilist[pl.ds(k * SEGF + j * CH, CH)]
            return pltpu.make_async_copy(
                tables[k].at[idx16],
                rows_v.at[pl.ds(offs[k] + j * CH, CH), :],
                gsemA)

        def s_copy(k, j):
            idx16 = ilist[pl.ds(k * SEGF + j * CH, CH)]
            return pltpu.make_async_copy(
                rows_v.at[pl.ds(offs[k] + j * CH, CH), :],
                out_hbm.at[idx16],
                ssem)

        for k in range(4):
            lax.fori_loop(
                0, nch[k], lambda j, _, k=k: (g_copy(k, j).start(), 0)[1], 0)
        for k in range(4):
            lax.fori_loop(
                0, nch[k], lambda j, _, k=k: (g_copy(k, j).wait(), 0)[1], 0)
        for k in range(4):
            lax.fori_loop(
                0, nch[k], lambda j, _, k=k: (s_copy(k, j).start(), 0)[1], 0)
        for k in range(4):
            lax.fori_loop(
                0, nch[k], lambda j, _, k=k: (s_copy(k, j).wait(), 0)[1], 0)

    return run(out0, out1, out2, out3, cid1d)
